# baseline (speedup 1.0000x reference)
import functools

import jax
import jax.numpy as jnp
from jax import lax
from jax.experimental import pallas as pl
from jax.experimental.pallas import tpu as pltpu
from jax.experimental.pallas import tpu_sc as plsc

N = 10000
E = 320000
D = 128

NSUB = 32
NPW = 320
NPAD = NSUB * NPW
SHIFT = 16384
BLK = 2000
NBLK = E // BLK
CAP = 160
G = 128
RC = 64
NEG = float("-inf")
ER = 2500


def _tc_body(x_ref, w_ref, b_ref, a_ref, bm_ref):
    xb = x_ref[...]
    w1 = w_ref[0:D, :]
    w2 = w_ref[D : 2 * D, :]
    a_ref[...] = (
        jnp.dot(xb, w1 - w2, preferred_element_type=jnp.float32) + b_ref[...]
    )
    bm_ref[...] = jnp.dot(xb, w2, preferred_element_type=jnp.float32)


def _node_transforms(xp, W, b2):
    grid = NPAD // 1024
    return pl.pallas_call(
        _tc_body,
        grid=(grid,),
        in_specs=[
            pl.BlockSpec((1024, D), lambda i: (i, 0)),
            pl.BlockSpec((2 * D, D), lambda i: (0, 0)),
            pl.BlockSpec((1, D), lambda i: (0, 0)),
        ],
        out_specs=[
            pl.BlockSpec((1024, D), lambda i: (i, 0)),
            pl.BlockSpec((1024, D), lambda i: (i, 0)),
        ],
        out_shape=[
            jax.ShapeDtypeStruct((NPAD, D), jnp.float32),
            jax.ShapeDtypeStruct((NPAD, D), jnp.float32),
        ],
    )(xp, W, b2)


def _pack_body(s_ref, d_ref, e_ref):
    e_ref[...] = d_ref[...] * SHIFT + s_ref[...]


def _pack_edges(s2, d2):
    return pl.pallas_call(
        _pack_body,
        grid=(1,),
        in_specs=[
            pl.BlockSpec((ER, D), lambda i: (0, 0)),
            pl.BlockSpec((ER, D), lambda i: (0, 0)),
        ],
        out_specs=pl.BlockSpec((ER, D), lambda i: (0, 0)),
        out_shape=jax.ShapeDtypeStruct((ER, D), jnp.int32),
    )(s2, d2)


def _sc_kernel(enc_hbm, bm_hbm, a_hbm, out_hbm, encv, ebuf, gidx, rows, accf,
               astg, ostg, gsem):
    wid = lax.axis_index("s") * 2 + lax.axis_index("c")
    lo = wid * NPW
    elo = lo * SHIFT
    ehi = (lo + NPW) * SHIFT

    def init_acc(i, c):
        accf[pl.ds(i * 16, 16)] = jnp.full((16,), NEG, jnp.float32)
        return c

    lax.fori_loop(0, NPW * D // 16, init_acc, 0)
    for i in range(CAP // 16):
        ebuf[pl.ds(16 * i, 16)] = jnp.zeros((16,), jnp.int32)

    iota16 = lax.iota(jnp.int32, 16)

    def do_flush(n):
        for t in range(G // 16):
            gidx[pl.ds(16 * t, 16)] = ebuf[pl.ds(16 * t, 16)] & (SHIFT - 1)
        pltpu.async_copy(bm_hbm.at[gidx], rows, gsem).wait()

        def acc_body(r, c):
            ev = plsc.load_gather(ebuf, [jnp.full((16,), r, jnp.int32)])
            base = (lax.shift_right_logical(ev, 14) - lo) * D
            for j in range(D // 16):
                idx = base + (16 * j) + iota16
                cur = plsc.load_gather(accf, [idx])
                g = rows[r, pl.ds(16 * j, 16)]
                plsc.store_scatter(accf, [idx], jnp.maximum(cur, g))
            return c

        lax.fori_loop(0, n, acc_body, 0)

    def blk_body(bk, m):
        off = bk * BLK
        pltpu.sync_copy(enc_hbm.at[pl.ds(off, BLK)], encv)

        def vec_body(i, m):
            e = encv[pl.ds(i * 16, 16)]
            msk = (e >= elo) & (e < ehi)

            def hit(mm):
                cnt = jnp.cumsum(msk.astype(jnp.int32))
                plsc.store_scatter(ebuf, [mm + cnt - 1], e, mask=msk)
                m2 = mm + cnt[15]

                def fl(mmm):
                    do_flush(G)
                    ebuf[pl.ds(0, 16)] = ebuf[pl.ds(G, 16)]
                    return mmm - G

                return lax.cond(m2 >= G, fl, lambda mmm: mmm, m2)

            return lax.cond(jnp.any(msk), hit, lambda mm: mm, m)

        return lax.fori_loop(0, BLK // 16, vec_body, m)

    m_fin = lax.fori_loop(0, NBLK, blk_body, 0)
    do_flush(m_fin)

    for c in range(NPW // RC):
        pltpu.sync_copy(a_hbm.at[pl.ds(lo + c * RC, RC)], astg)

        def ep_body(r, cc):
            for j in range(D // 16):
                v = astg[r, pl.ds(16 * j, 16)] + accf[
                    pl.ds((c * RC + r) * D + 16 * j, 16)
                ]
                ostg[r, pl.ds(16 * j, 16)] = jnp.maximum(v, 0.0)
            return cc

        lax.fori_loop(0, RC, ep_body, 0)
        pltpu.sync_copy(ostg, out_hbm.at[pl.ds(lo + c * RC, RC)])


_sc_call = functools.partial(
    pl.kernel,
    mesh=plsc.VectorSubcoreMesh(core_axis_name="c", subcore_axis_name="s"),
    out_type=jax.ShapeDtypeStruct((NPAD, D), jnp.float32),
    scratch_types=[
        pltpu.VMEM((BLK,), jnp.int32),
        pltpu.VMEM((CAP,), jnp.int32),
        pltpu.VMEM((G,), jnp.int32),
        pltpu.VMEM((G, D), jnp.float32),
        pltpu.VMEM((NPW * D,), jnp.float32),
        pltpu.VMEM((RC, D), jnp.float32),
        pltpu.VMEM((RC, D), jnp.float32),
        pltpu.SemaphoreType.DMA,
    ],
    compiler_params=pltpu.CompilerParams(needs_layout_passes=False),
)(_sc_kernel)


@jax.jit
def kernel(x, edge_index, W, b):
    xp = jnp.zeros((NPAD, D), jnp.float32).at[:N].set(x)
    s2 = edge_index[0].reshape(ER, D)
    d2 = edge_index[1].reshape(ER, D)
    A, Bm = _node_transforms(xp, W, b.reshape(1, D))
    enc2 = _pack_edges(s2, d2)
    outp = _sc_call(enc2.reshape(E), Bm, A)
    return outp[:N]


# speedup vs baseline: 1.2419x; 1.2419x over previous
import functools

import jax
import jax.numpy as jnp
from jax import lax
from jax.experimental import pallas as pl
from jax.experimental.pallas import tpu as pltpu
from jax.experimental.pallas import tpu_sc as plsc

N = 10000
E = 320000
D = 128

NSUB = 32
NPW = 320
NPAD = NSUB * NPW
SHIFT = 16384
BLK = 2000
NBLK = E // BLK
CAP = 160
G = 128
RC = 64
NEG = float("-inf")
ER = 2500


def _tc_body(x_ref, w_ref, b_ref, a_ref, bm_ref):
    xb = x_ref[...]
    w1 = w_ref[0:D, :]
    w2 = w_ref[D : 2 * D, :]
    a_ref[...] = (
        jnp.dot(xb, w1 - w2, preferred_element_type=jnp.float32) + b_ref[...]
    )
    bm_ref[...] = jnp.dot(xb, w2, preferred_element_type=jnp.float32)


def _node_transforms(xp, W, b2):
    grid = NPAD // 1024
    return pl.pallas_call(
        _tc_body,
        grid=(grid,),
        in_specs=[
            pl.BlockSpec((1024, D), lambda i: (i, 0)),
            pl.BlockSpec((2 * D, D), lambda i: (0, 0)),
            pl.BlockSpec((1, D), lambda i: (0, 0)),
        ],
        out_specs=[
            pl.BlockSpec((1024, D), lambda i: (i, 0)),
            pl.BlockSpec((1024, D), lambda i: (i, 0)),
        ],
        out_shape=[
            jax.ShapeDtypeStruct((NPAD, D), jnp.float32),
            jax.ShapeDtypeStruct((NPAD, D), jnp.float32),
        ],
    )(xp, W, b2)


def _pack_body(s_ref, d_ref, e_ref):
    e_ref[...] = d_ref[...] * SHIFT + s_ref[...]


def _pack_edges(s2, d2):
    return pl.pallas_call(
        _pack_body,
        grid=(1,),
        in_specs=[
            pl.BlockSpec((ER, D), lambda i: (0, 0)),
            pl.BlockSpec((ER, D), lambda i: (0, 0)),
        ],
        out_specs=pl.BlockSpec((ER, D), lambda i: (0, 0)),
        out_shape=jax.ShapeDtypeStruct((ER, D), jnp.int32),
    )(s2, d2)


def _sc_kernel(enc_hbm, bm_hbm, a_hbm, out_hbm, encv, ebuf, gidx, rows, accf,
               astg, ostg, gsem):
    wid = lax.axis_index("s") * 2 + lax.axis_index("c")
    lo = wid * NPW
    elo = lo * SHIFT
    ehi = (lo + NPW) * SHIFT

    def init_acc(i, c):
        accf[pl.ds(i * 16, 16)] = jnp.full((16,), NEG, jnp.float32)
        return c

    lax.fori_loop(0, NPW * D // 16, init_acc, 0)
    for i in range(CAP // 16):
        ebuf[pl.ds(16 * i, 16)] = jnp.zeros((16,), jnp.int32)

    iota16 = lax.iota(jnp.int32, 16)

    def do_flush(n):
        for t in range(G // 16):
            gidx[pl.ds(16 * t, 16)] = ebuf[pl.ds(16 * t, 16)] & (SHIFT - 1)
        pltpu.async_copy(bm_hbm.at[gidx], rows, gsem).wait()

        def acc_body(r, c):
            ev = plsc.load_gather(ebuf, [jnp.full((16,), r, jnp.int32)])
            base = (lax.shift_right_logical(ev, 14) - lo) * D
            for j in range(D // 16):
                idx = base + (16 * j) + iota16
                cur = plsc.load_gather(accf, [idx])
                g = rows[r, pl.ds(16 * j, 16)]
                plsc.store_scatter(accf, [idx], jnp.maximum(cur, g))
            return c

        lax.fori_loop(0, n, acc_body, 0)

    def blk_body(bk, m):
        off = bk * BLK
        pltpu.sync_copy(enc_hbm.at[pl.ds(off, BLK)], encv)

        def vec_body(i, m):
            e = encv[pl.ds(i * 16, 16)]
            msk = (e >= elo) & (e < ehi)
            cnt = jnp.cumsum(msk.astype(jnp.int32))
            plsc.store_scatter(ebuf, [m + cnt - 1], e, mask=msk)
            m2 = m + cnt[15]

            def fl(mm):
                do_flush(G)
                ebuf[pl.ds(0, 16)] = ebuf[pl.ds(G, 16)]
                return mm - G

            return lax.cond(m2 >= G, fl, lambda mm: mm, m2)

        return lax.fori_loop(0, BLK // 16, vec_body, m)

    m_fin = lax.fori_loop(0, NBLK, blk_body, 0)
    do_flush(m_fin)

    for c in range(NPW // RC):
        pltpu.sync_copy(a_hbm.at[pl.ds(lo + c * RC, RC)], astg)

        def ep_body(r, cc):
            for j in range(D // 16):
                v = astg[r, pl.ds(16 * j, 16)] + accf[
                    pl.ds((c * RC + r) * D + 16 * j, 16)
                ]
                ostg[r, pl.ds(16 * j, 16)] = jnp.maximum(v, 0.0)
            return cc

        lax.fori_loop(0, RC, ep_body, 0)
        pltpu.sync_copy(ostg, out_hbm.at[pl.ds(lo + c * RC, RC)])


_sc_call = functools.partial(
    pl.kernel,
    mesh=plsc.VectorSubcoreMesh(core_axis_name="c", subcore_axis_name="s"),
    out_type=jax.ShapeDtypeStruct((NPAD, D), jnp.float32),
    scratch_types=[
        pltpu.VMEM((BLK,), jnp.int32),
        pltpu.VMEM((CAP,), jnp.int32),
        pltpu.VMEM((G,), jnp.int32),
        pltpu.VMEM((G, D), jnp.float32),
        pltpu.VMEM((NPW * D,), jnp.float32),
        pltpu.VMEM((RC, D), jnp.float32),
        pltpu.VMEM((RC, D), jnp.float32),
        pltpu.SemaphoreType.DMA,
    ],
    compiler_params=pltpu.CompilerParams(needs_layout_passes=False),
)(_sc_kernel)


@jax.jit
def kernel(x, edge_index, W, b):
    xp = jnp.zeros((NPAD, D), jnp.float32).at[:N].set(x)
    s2 = edge_index[0].reshape(ER, D)
    d2 = edge_index[1].reshape(ER, D)
    A, Bm = _node_transforms(xp, W, b.reshape(1, D))
    enc2 = _pack_edges(s2, d2)
    outp = _sc_call(enc2.reshape(E), Bm, A)
    return outp[:N]


# speedup vs baseline: 1.5964x; 1.2854x over previous
import functools

import jax
import jax.numpy as jnp
from jax import lax
from jax.experimental import pallas as pl
from jax.experimental.pallas import tpu as pltpu
from jax.experimental.pallas import tpu_sc as plsc

N = 10000
E = 320000
D = 128

NSUB = 32
NPW = 320
NPAD = NSUB * NPW
SHIFT = 16384
BLK = 4000
NBLK = E // BLK
CAP = 160
G = 128
RC = 64
NEG = float("-inf")
ER = 2500


def _tc_body(x_ref, w_ref, b_ref, a_ref, bm_ref):
    xb = x_ref[...]
    w1 = w_ref[0:D, :]
    w2 = w_ref[D : 2 * D, :]
    a_ref[...] = (
        jnp.dot(xb, w1 - w2, preferred_element_type=jnp.float32) + b_ref[...]
    )
    bm_ref[...] = jnp.dot(xb, w2, preferred_element_type=jnp.float32)


def _node_transforms(xp, W, b2):
    grid = NPAD // 1024
    return pl.pallas_call(
        _tc_body,
        grid=(grid,),
        in_specs=[
            pl.BlockSpec((1024, D), lambda i: (i, 0)),
            pl.BlockSpec((2 * D, D), lambda i: (0, 0)),
            pl.BlockSpec((1, D), lambda i: (0, 0)),
        ],
        out_specs=[
            pl.BlockSpec((1024, D), lambda i: (i, 0)),
            pl.BlockSpec((1024, D), lambda i: (i, 0)),
        ],
        out_shape=[
            jax.ShapeDtypeStruct((NPAD, D), jnp.float32),
            jax.ShapeDtypeStruct((NPAD, D), jnp.float32),
        ],
    )(xp, W, b2)


def _pack_body(s_ref, d_ref, e_ref):
    e_ref[...] = d_ref[...] * SHIFT + s_ref[...]


def _pack_edges(s2, d2):
    return pl.pallas_call(
        _pack_body,
        grid=(1,),
        in_specs=[
            pl.BlockSpec((ER, D), lambda i: (0, 0)),
            pl.BlockSpec((ER, D), lambda i: (0, 0)),
        ],
        out_specs=pl.BlockSpec((ER, D), lambda i: (0, 0)),
        out_shape=jax.ShapeDtypeStruct((ER, D), jnp.int32),
    )(s2, d2)


def _sc_kernel(enc_hbm, bm_hbm, a_hbm, out_hbm, encv, ebuf, gidx, rows, accf,
               astg, ostg, gsem):
    wid = lax.axis_index("s") * 2 + lax.axis_index("c")
    lo = wid * NPW
    elo = lo * SHIFT
    ehi = (lo + NPW) * SHIFT

    def init_acc(i, c):
        accf[pl.ds(i * 16, 16)] = jnp.full((16,), NEG, jnp.float32)
        return c

    lax.fori_loop(0, NPW * D // 16, init_acc, 0)
    for i in range(CAP // 16):
        ebuf[pl.ds(16 * i, 16)] = jnp.zeros((16,), jnp.int32)

    iota16 = lax.iota(jnp.int32, 16)

    def do_flush(n):
        for t in range(G // 16):
            gidx[pl.ds(16 * t, 16)] = ebuf[pl.ds(16 * t, 16)] & (SHIFT - 1)
        pltpu.async_copy(bm_hbm.at[gidx], rows, gsem).wait()

        def acc_body(r, c):
            ev = plsc.load_gather(ebuf, [jnp.full((16,), r, jnp.int32)])
            base = (lax.shift_right_logical(ev, 14) - lo) * D
            for j in range(D // 16):
                idx = base + (16 * j) + iota16
                cur = plsc.load_gather(accf, [idx])
                g = rows[r, pl.ds(16 * j, 16)]
                plsc.store_scatter(accf, [idx], jnp.maximum(cur, g))
            return c

        lax.fori_loop(0, n, acc_body, 0)

    def blk_body(bk, m):
        off = bk * BLK
        pltpu.sync_copy(enc_hbm.at[pl.ds(off, BLK)], encv)

        def vec_body(i, m):
            e1 = encv[pl.ds(i * 32, 16)]
            e2 = encv[pl.ds(i * 32 + 16, 16)]
            msk1 = (e1 >= elo) & (e1 < ehi)
            msk2 = (e2 >= elo) & (e2 < ehi)
            cnt1 = jnp.cumsum(msk1.astype(jnp.int32))
            cnt2 = jnp.cumsum(msk2.astype(jnp.int32))
            plsc.store_scatter(ebuf, [m + cnt1 - 1], e1, mask=msk1)
            t1 = m + cnt1[15]
            plsc.store_scatter(ebuf, [t1 + cnt2 - 1], e2, mask=msk2)
            m2 = t1 + cnt2[15]

            def fl(mm):
                do_flush(G)
                ebuf[pl.ds(0, 16)] = ebuf[pl.ds(G, 16)]
                ebuf[pl.ds(16, 16)] = ebuf[pl.ds(G + 16, 16)]
                return mm - G

            return lax.cond(m2 >= G, fl, lambda mm: mm, m2)

        return lax.fori_loop(0, BLK // 32, vec_body, m)

    m_fin = lax.fori_loop(0, NBLK, blk_body, 0)
    do_flush(m_fin)

    for c in range(NPW // RC):
        pltpu.sync_copy(a_hbm.at[pl.ds(lo + c * RC, RC)], astg)

        def ep_body(r, cc):
            for j in range(D // 16):
                v = astg[r, pl.ds(16 * j, 16)] + accf[
                    pl.ds((c * RC + r) * D + 16 * j, 16)
                ]
                ostg[r, pl.ds(16 * j, 16)] = jnp.maximum(v, 0.0)
            return cc

        lax.fori_loop(0, RC, ep_body, 0)
        pltpu.sync_copy(ostg, out_hbm.at[pl.ds(lo + c * RC, RC)])


_sc_call = functools.partial(
    pl.kernel,
    mesh=plsc.VectorSubcoreMesh(core_axis_name="c", subcore_axis_name="s"),
    out_type=jax.ShapeDtypeStruct((NPAD, D), jnp.float32),
    scratch_types=[
        pltpu.VMEM((BLK,), jnp.int32),
        pltpu.VMEM((CAP,), jnp.int32),
        pltpu.VMEM((G,), jnp.int32),
        pltpu.VMEM((G, D), jnp.float32),
        pltpu.VMEM((NPW * D,), jnp.float32),
        pltpu.VMEM((RC, D), jnp.float32),
        pltpu.VMEM((RC, D), jnp.float32),
        pltpu.SemaphoreType.DMA,
    ],
    compiler_params=pltpu.CompilerParams(needs_layout_passes=False),
)(_sc_kernel)


@jax.jit
def kernel(x, edge_index, W, b):
    xp = jnp.zeros((NPAD, D), jnp.float32).at[:N].set(x)
    s2 = edge_index[0].reshape(ER, D)
    d2 = edge_index[1].reshape(ER, D)
    A, Bm = _node_transforms(xp, W, b.reshape(1, D))
    enc2 = _pack_edges(s2, d2)
    outp = _sc_call(enc2.reshape(E), Bm, A)
    return outp[:N]


# speedup vs baseline: 1.6096x; 1.0083x over previous
import functools

import jax
import jax.numpy as jnp
from jax import lax
from jax.experimental import pallas as pl
from jax.experimental.pallas import tpu as pltpu
from jax.experimental.pallas import tpu_sc as plsc

N = 10000
E = 320000
D = 128

NSUB = 32
NPW = 320
NPAD = NSUB * NPW
SHIFT = 16384
BLK = 4000
NBLK = E // BLK
CAP = 160
G = 128
RC = 64
NEG = float("-inf")
ER = 2500


def _tc_body(x_ref, w_ref, b_ref, a_ref, bm_ref):
    xb = x_ref[...]
    w1 = w_ref[0:D, :]
    w2 = w_ref[D : 2 * D, :]
    a_ref[...] = (
        jnp.dot(xb, w1 - w2, preferred_element_type=jnp.float32) + b_ref[...]
    )
    bm_ref[...] = jnp.dot(xb, w2, preferred_element_type=jnp.float32)


def _node_transforms(xp, W, b2):
    grid = NPAD // 1024
    return pl.pallas_call(
        _tc_body,
        grid=(grid,),
        in_specs=[
            pl.BlockSpec((1024, D), lambda i: (i, 0)),
            pl.BlockSpec((2 * D, D), lambda i: (0, 0)),
            pl.BlockSpec((1, D), lambda i: (0, 0)),
        ],
        out_specs=[
            pl.BlockSpec((1024, D), lambda i: (i, 0)),
            pl.BlockSpec((1024, D), lambda i: (i, 0)),
        ],
        out_shape=[
            jax.ShapeDtypeStruct((NPAD, D), jnp.float32),
            jax.ShapeDtypeStruct((NPAD, D), jnp.float32),
        ],
    )(xp, W, b2)


def _pack_body(s_ref, d_ref, e_ref):
    e_ref[...] = d_ref[...] * SHIFT + s_ref[...]


def _pack_edges(s2, d2):
    return pl.pallas_call(
        _pack_body,
        grid=(1,),
        in_specs=[
            pl.BlockSpec((ER, D), lambda i: (0, 0)),
            pl.BlockSpec((ER, D), lambda i: (0, 0)),
        ],
        out_specs=pl.BlockSpec((ER, D), lambda i: (0, 0)),
        out_shape=jax.ShapeDtypeStruct((ER, D), jnp.int32),
    )(s2, d2)


def _sc_kernel(enc_hbm, bm_hbm, a_hbm, out_hbm, encv, ebuf, gidxa, gidxb,
               rows, accf, astg, ostg, gsem, gsem2):
    wid = lax.axis_index("s") * 2 + lax.axis_index("c")
    lo = wid * NPW
    elo = lo * SHIFT
    ehi = (lo + NPW) * SHIFT

    def init_acc(i, c):
        accf[pl.ds(i * 16, 16)] = jnp.full((16,), NEG, jnp.float32)
        return c

    lax.fori_loop(0, NPW * D // 16, init_acc, 0)
    for i in range(CAP // 16):
        ebuf[pl.ds(16 * i, 16)] = jnp.zeros((16,), jnp.int32)

    iota16 = lax.iota(jnp.int32, 16)

    def do_flush(n):
        for t in range(G // 32):
            gidxa[pl.ds(16 * t, 16)] = ebuf[pl.ds(16 * t, 16)] & (SHIFT - 1)
            gidxb[pl.ds(16 * t, 16)] = ebuf[pl.ds(64 + 16 * t, 16)] & (SHIFT - 1)
        ca = pltpu.async_copy(bm_hbm.at[gidxa], rows.at[pl.ds(0, 64)], gsem)
        cb = pltpu.async_copy(bm_hbm.at[gidxb], rows.at[pl.ds(64, 64)], gsem2)

        def acc_body(r, c):
            ev = plsc.load_gather(ebuf, [jnp.full((16,), r, jnp.int32)])
            base = (lax.shift_right_logical(ev, 14) - lo) * D
            for j in range(D // 16):
                idx = base + (16 * j) + iota16
                cur = plsc.load_gather(accf, [idx])
                g = rows[r, pl.ds(16 * j, 16)]
                plsc.store_scatter(accf, [idx], jnp.maximum(cur, g))
            return c

        ca.wait()
        lax.fori_loop(0, jnp.minimum(n, 64), acc_body, 0)
        cb.wait()
        lax.fori_loop(64, jnp.maximum(n, 64), acc_body, 0)

    def blk_body(bk, m):
        off = bk * BLK
        pltpu.sync_copy(enc_hbm.at[pl.ds(off, BLK)], encv)

        def vec_body(i, m):
            e1 = encv[pl.ds(i * 32, 16)]
            e2 = encv[pl.ds(i * 32 + 16, 16)]
            msk1 = (e1 >= elo) & (e1 < ehi)
            msk2 = (e2 >= elo) & (e2 < ehi)
            cnt1 = jnp.cumsum(msk1.astype(jnp.int32))
            cnt2 = jnp.cumsum(msk2.astype(jnp.int32))
            plsc.store_scatter(ebuf, [m + cnt1 - 1], e1, mask=msk1)
            t1 = m + cnt1[15]
            plsc.store_scatter(ebuf, [t1 + cnt2 - 1], e2, mask=msk2)
            m2 = t1 + cnt2[15]

            def fl(mm):
                do_flush(G)
                ebuf[pl.ds(0, 16)] = ebuf[pl.ds(G, 16)]
                ebuf[pl.ds(16, 16)] = ebuf[pl.ds(G + 16, 16)]
                return mm - G

            return lax.cond(m2 >= G, fl, lambda mm: mm, m2)

        return lax.fori_loop(0, BLK // 32, vec_body, m)

    m_fin = lax.fori_loop(0, NBLK, blk_body, 0)
    do_flush(m_fin)

    for c in range(NPW // RC):
        pltpu.sync_copy(a_hbm.at[pl.ds(lo + c * RC, RC)], astg)

        def ep_body(r, cc):
            for j in range(D // 16):
                v = astg[r, pl.ds(16 * j, 16)] + accf[
                    pl.ds((c * RC + r) * D + 16 * j, 16)
                ]
                ostg[r, pl.ds(16 * j, 16)] = jnp.maximum(v, 0.0)
            return cc

        lax.fori_loop(0, RC, ep_body, 0)
        pltpu.sync_copy(ostg, out_hbm.at[pl.ds(lo + c * RC, RC)])


_sc_call = functools.partial(
    pl.kernel,
    mesh=plsc.VectorSubcoreMesh(core_axis_name="c", subcore_axis_name="s"),
    out_type=jax.ShapeDtypeStruct((NPAD, D), jnp.float32),
    scratch_types=[
        pltpu.VMEM((BLK,), jnp.int32),
        pltpu.VMEM((CAP,), jnp.int32),
        pltpu.VMEM((G // 2,), jnp.int32),
        pltpu.VMEM((G // 2,), jnp.int32),
        pltpu.VMEM((G, D), jnp.float32),
        pltpu.VMEM((NPW * D,), jnp.float32),
        pltpu.VMEM((RC, D), jnp.float32),
        pltpu.VMEM((RC, D), jnp.float32),
        pltpu.SemaphoreType.DMA,
        pltpu.SemaphoreType.DMA,
    ],
    compiler_params=pltpu.CompilerParams(needs_layout_passes=False),
)(_sc_kernel)


@jax.jit
def kernel(x, edge_index, W, b):
    xp = jnp.zeros((NPAD, D), jnp.float32).at[:N].set(x)
    s2 = edge_index[0].reshape(ER, D)
    d2 = edge_index[1].reshape(ER, D)
    A, Bm = _node_transforms(xp, W, b.reshape(1, D))
    enc2 = _pack_edges(s2, d2)
    outp = _sc_call(enc2.reshape(E), Bm, A)
    return outp[:N]
